# baseline (speedup 1.0000x reference)
import jax
import jax.numpy as jnp
from jax import lax
from jax.experimental import pallas as pl
from jax.experimental.pallas import tpu as pltpu
from jax.experimental.pallas import tpu_sc as plsc

VOCAB = 1000000
DIM = 32
B = 4096
L = 200

_INFO = plsc.get_sparse_core_info()
_NC, _NS = _INFO.num_cores, _INFO.num_subcores
_NW = _NC * _NS
_BATCH = 128
_NCHUNK = 8
_LPC = L // _NCHUNK


def _gather_tr_body(ids_hbm, table_hbm, re_hbm, im_hbm, idx_v, rows_v, tr_v, sem):
    wid = lax.axis_index("s") * _NC + lax.axis_index("c")
    b0 = wid * _BATCH
    pltpu.sync_copy(ids_hbm.at[:, pl.ds(b0, _BATCH)], idx_v)

    def lstep(l, carry):
        pltpu.async_copy(table_hbm.at[idx_v.at[l]], rows_v, sem).wait()

        def tstep(j, carry2):
            d = j // 8
            k = j % 8
            row_idx = k * 16 + lax.iota(jnp.int32, 16)
            col_idx = lax.iota(jnp.int32, 16) * 0 + d
            v = plsc.load_gather(rows_v, [row_idx, col_idx])
            tr_v[d, pl.ds(k * 16, 16)] = v
            return carry2

        lax.fori_loop(0, 2 * DIM * 8, tstep, 0)

        pltpu.sync_copy(tr_v.at[pl.ds(0, DIM)], re_hbm.at[l, :, pl.ds(b0, _BATCH)])
        pltpu.sync_copy(tr_v.at[pl.ds(DIM, DIM)], im_hbm.at[l, :, pl.ds(b0, _BATCH)])
        return carry

    lax.fori_loop(0, _LPC, lstep, 0)


def _sc_chunk(ids_t_chunk, table):
    mesh = plsc.VectorSubcoreMesh(core_axis_name="c", subcore_axis_name="s")
    return pl.kernel(
        _gather_tr_body,
        out_type=(
            jax.ShapeDtypeStruct((_LPC, DIM, B), jnp.float32),
            jax.ShapeDtypeStruct((_LPC, DIM, B), jnp.float32),
        ),
        mesh=mesh,
        scratch_types=[
            pltpu.VMEM((_LPC, _BATCH), jnp.int32),
            pltpu.VMEM((_BATCH, 2 * DIM), jnp.float32),
            pltpu.VMEM((2 * DIM, _BATCH), jnp.float32),
            pltpu.SemaphoreType.DMA,
        ],
        compiler_params=pltpu.CompilerParams(use_tc_tiling_on_sc=False, needs_layout_passes=False),
    )(ids_t_chunk, table)


def kernel(ids, table):
    ids_t = ids.T
    outs = []
    for i in range(_NCHUNK):
        re_t, im_t = _sc_chunk(ids_t[i * _LPC : (i + 1) * _LPC], table)
        re = re_t.transpose(2, 0, 1)
        im = im_t.transpose(2, 0, 1)
        outs.append(lax.complex(re, im))
    return jnp.concatenate(outs, axis=1)


# speedup vs baseline: 1.0885x; 1.0885x over previous
import jax
import jax.numpy as jnp
from jax import lax
from jax.experimental import pallas as pl
from jax.experimental.pallas import tpu as pltpu
from jax.experimental.pallas import tpu_sc as plsc

VOCAB = 1000000
DIM = 32
B = 4096
L = 200

_INFO = plsc.get_sparse_core_info()
_NC, _NS = _INFO.num_cores, _INFO.num_subcores
_NW = _NC * _NS
_BATCH = 128
_NCHUNK = 8
_LPC = L // _NCHUNK


def _gather_tr_body(ids_hbm, table_hbm, re_hbm, im_hbm, idx_v, rows_v, tr_v, sem):
    wid = lax.axis_index("s") * _NC + lax.axis_index("c")
    b0 = wid * _BATCH
    pltpu.sync_copy(ids_hbm.at[:, pl.ds(b0, _BATCH)], idx_v)

    def lstep(l, carry):
        pltpu.async_copy(table_hbm.at[idx_v.at[l]], rows_v, sem).wait()

        @plsc.parallel_loop(0, 2 * DIM * 8, unroll=8)
        def tstep(j):
            d = j // 8
            k = j % 8
            row_idx = k * 16 + lax.iota(jnp.int32, 16)
            col_idx = lax.iota(jnp.int32, 16) * 0 + d
            v = plsc.load_gather(rows_v, [row_idx, col_idx])
            tr_v[d, pl.ds(k * 16, 16)] = v

        pltpu.sync_copy(tr_v.at[pl.ds(0, DIM)], re_hbm.at[l, :, pl.ds(b0, _BATCH)])
        pltpu.sync_copy(tr_v.at[pl.ds(DIM, DIM)], im_hbm.at[l, :, pl.ds(b0, _BATCH)])
        return carry

    lax.fori_loop(0, _LPC, lstep, 0)


def _sc_chunk(ids_t_chunk, table):
    mesh = plsc.VectorSubcoreMesh(core_axis_name="c", subcore_axis_name="s")
    return pl.kernel(
        _gather_tr_body,
        out_type=(
            jax.ShapeDtypeStruct((_LPC, DIM, B), jnp.float32),
            jax.ShapeDtypeStruct((_LPC, DIM, B), jnp.float32),
        ),
        mesh=mesh,
        scratch_types=[
            pltpu.VMEM((_LPC, _BATCH), jnp.int32),
            pltpu.VMEM((_BATCH, 2 * DIM), jnp.float32),
            pltpu.VMEM((2 * DIM, _BATCH), jnp.float32),
            pltpu.SemaphoreType.DMA,
        ],
        compiler_params=pltpu.CompilerParams(use_tc_tiling_on_sc=False, needs_layout_passes=False),
    )(ids_t_chunk, table)


def kernel(ids, table):
    ids_t = ids.T
    outs = []
    for i in range(_NCHUNK):
        idc = ids_t[i * _LPC : (i + 1) * _LPC]
        if i >= 2:
            idc, _ = lax.optimization_barrier((idc, outs[i - 2]))
        re_t, im_t = _sc_chunk(idc, table)
        re = re_t.transpose(2, 0, 1)
        im = im_t.transpose(2, 0, 1)
        outs.append(lax.complex(re, im))
    return jnp.concatenate(outs, axis=1)


# speedup vs baseline: 1.1645x; 1.0698x over previous
import jax
import jax.numpy as jnp
from jax import lax
from jax.experimental import pallas as pl
from jax.experimental.pallas import tpu as pltpu
from jax.experimental.pallas import tpu_sc as plsc

VOCAB = 1000000
DIM = 32
B = 4096
L = 200

_INFO = plsc.get_sparse_core_info()
_NC, _NS = _INFO.num_cores, _INFO.num_subcores
_NW = _NC * _NS
_BATCH = 128
_NCHUNK = 8
_LPC = L // _NCHUNK


def _gather_tr_body(ids_hbm, table_hbm, re_hbm, im_hbm, idx_v, rows_v, tr_v, sem):
    wid = lax.axis_index("s") * _NC + lax.axis_index("c")
    b0 = wid * _BATCH
    pltpu.sync_copy(ids_hbm.at[:, pl.ds(b0, _BATCH)], idx_v)

    def lstep(l, carry):
        pltpu.async_copy(table_hbm.at[idx_v.at[l]], rows_v, sem).wait()

        iota = lax.iota(jnp.int32, 16)

        @plsc.parallel_loop(0, 2 * DIM, unroll=4)
        def tstep(d):
            col_idx = iota * 0 + d
            for k in range(8):
                v = plsc.load_gather(rows_v, [k * 16 + iota, col_idx])
                tr_v[d, pl.ds(k * 16, 16)] = v

        pltpu.sync_copy(tr_v.at[pl.ds(0, DIM)], re_hbm.at[l, :, pl.ds(b0, _BATCH)])
        pltpu.sync_copy(tr_v.at[pl.ds(DIM, DIM)], im_hbm.at[l, :, pl.ds(b0, _BATCH)])
        return carry

    lax.fori_loop(0, _LPC, lstep, 0)


def _sc_chunk(ids_t_chunk, table):
    mesh = plsc.VectorSubcoreMesh(core_axis_name="c", subcore_axis_name="s")
    return pl.kernel(
        _gather_tr_body,
        out_type=(
            jax.ShapeDtypeStruct((_LPC, DIM, B), jnp.float32),
            jax.ShapeDtypeStruct((_LPC, DIM, B), jnp.float32),
        ),
        mesh=mesh,
        scratch_types=[
            pltpu.VMEM((_LPC, _BATCH), jnp.int32),
            pltpu.VMEM((_BATCH, 2 * DIM), jnp.float32),
            pltpu.VMEM((2 * DIM, _BATCH), jnp.float32),
            pltpu.SemaphoreType.DMA,
        ],
        compiler_params=pltpu.CompilerParams(use_tc_tiling_on_sc=False, needs_layout_passes=False),
    )(ids_t_chunk, table)


def kernel(ids, table):
    ids_t = ids.T
    outs = []
    for i in range(_NCHUNK):
        idc = ids_t[i * _LPC : (i + 1) * _LPC]
        if i >= 2:
            idc, _ = lax.optimization_barrier((idc, outs[i - 2]))
        re_t, im_t = _sc_chunk(idc, table)
        re = re_t.transpose(2, 0, 1)
        im = im_t.transpose(2, 0, 1)
        outs.append(lax.complex(re, im))
    return jnp.concatenate(outs, axis=1)


# speedup vs baseline: 1.2012x; 1.0315x over previous
import jax
import jax.numpy as jnp
from jax import lax
from jax.experimental import pallas as pl
from jax.experimental.pallas import tpu as pltpu
from jax.experimental.pallas import tpu_sc as plsc

VOCAB = 1000000
DIM = 32
B = 4096
L = 200

_INFO = plsc.get_sparse_core_info()
_NC, _NS = _INFO.num_cores, _INFO.num_subcores
_NW = _NC * _NS
_BATCH = 128
_NCHUNK = 8
_LPC = L // _NCHUNK


def _gather_tr_body(ids_hbm, table_hbm, re_hbm, im_hbm, idx_v, rows_v, tr_v, sem):
    wid = lax.axis_index("s") * _NC + lax.axis_index("c")
    b0 = wid * _BATCH
    pltpu.sync_copy(ids_hbm.at[:, pl.ds(b0, _BATCH)], idx_v)

    def lstep(l, carry):
        pltpu.async_copy(table_hbm.at[idx_v.at[l]], rows_v, sem).wait()

        iota = lax.iota(jnp.int32, 16)

        @plsc.parallel_loop(0, 2 * DIM, unroll=4)
        def tstep(d):
            col_idx = iota * 0 + d
            for k in range(8):
                v = plsc.load_gather(rows_v, [k * 16 + iota, col_idx])
                tr_v[d, pl.ds(k * 16, 16)] = v

        pltpu.sync_copy(tr_v.at[pl.ds(0, DIM)], re_hbm.at[l, :, pl.ds(b0, _BATCH)])
        pltpu.sync_copy(tr_v.at[pl.ds(DIM, DIM)], im_hbm.at[l, :, pl.ds(b0, _BATCH)])
        return carry

    lax.fori_loop(0, _LPC, lstep, 0)


def _sc_chunk(ids_t_chunk, table):
    mesh = plsc.VectorSubcoreMesh(core_axis_name="c", subcore_axis_name="s")
    return pl.kernel(
        _gather_tr_body,
        out_type=(
            jax.ShapeDtypeStruct((_LPC, DIM, B), jnp.float32),
            jax.ShapeDtypeStruct((_LPC, DIM, B), jnp.float32),
        ),
        mesh=mesh,
        scratch_types=[
            pltpu.VMEM((_LPC, _BATCH), jnp.int32),
            pltpu.VMEM((_BATCH, 2 * DIM), jnp.float32),
            pltpu.VMEM((2 * DIM, _BATCH), jnp.float32),
            pltpu.SemaphoreType.DMA,
        ],
        compiler_params=pltpu.CompilerParams(use_tc_tiling_on_sc=False, needs_layout_passes=False),
    )(ids_t_chunk, table)


def kernel(ids, table):
    ids_t = ids.T
    outs = []
    for i in range(_NCHUNK):
        idc = ids_t[i * _LPC : (i + 1) * _LPC]
        if i >= 2:
            idc, _ = lax.optimization_barrier((idc, outs[i - 2]))
        re_t, im_t = _sc_chunk(idc, table)
        re = re_t.transpose(2, 0, 1)
        im = im_t.transpose(2, 0, 1)
        c = lax.complex(re, im)
        (c,) = lax.optimization_barrier((c,))
        outs.append(c)
    return jnp.concatenate(outs, axis=1)
